# baseline (speedup 1.0000x reference)
import functools

import jax
import jax.numpy as jnp
from jax import lax
from jax.experimental import pallas as pl
from jax.experimental.pallas import tpu as pltpu
from jax.experimental.pallas import tpu_sc as plsc

B, T, TDIM = 4096, 200, 64
N = B * T
NC, NS, L = 2, 16, 16
NW = NC * NS
NPW = N // NW
CHUNK = 128
NCHUNK = NPW // CHUNK
GROUPS = CHUNK // L
TEROWS = CHUNK * 5 // 128
NDAY = 288


def _body(te_hbm, comb_hbm, out_hbm,
          comb_t, te_v, cidx_v, rows_v,
          tsem, gsem, osem, lsem):
    sid = lax.axis_index("s")
    wid = sid * NC + lax.axis_index("c")
    lane5 = jax.lax.iota(jnp.int32, L) * 5
    chunk0 = wid * NCHUNK

    @pl.when(sid == 0)
    def _load_table():
        pltpu.async_copy(comb_hbm, comb_t, lsem).wait()

    plsc.subcore_barrier()

    def te_copy(k, slot):
        return pltpu.make_async_copy(
            te_hbm.at[pl.ds((chunk0 + k) * TEROWS, TEROWS)],
            te_v.at[slot], tsem.at[slot])

    def gather(slot):
        return pltpu.make_async_copy(
            comb_hbm.at[cidx_v.at[slot]], rows_v.at[slot], gsem.at[slot])

    def out_copy(k, slot):
        return pltpu.make_async_copy(
            rows_v.at[slot],
            out_hbm.at[pl.ds((chunk0 + k) * CHUNK, CHUNK)], osem.at[slot])

    te_copy(0, 0).start()

    def chunk_step(k, slot):
        te_copy(k, slot).wait()

        @pl.when(k + 1 < NCHUNK)
        def _prefetch():
            te_copy(k + 1, slot ^ 1).start()

        @pl.when(k >= 2)
        def _drain_out():
            out_copy(k - 2, slot).wait()

        @plsc.parallel_loop(0, GROUPS, unroll=2)
        def group_step(g):
            i5 = lane5 + g * (L * 5)

            def col(off):
                idx = i5 + off
                return plsc.load_gather(te_v.at[slot], [idx >> 7, idx & 127])

            w = col(2)
            h = col(3)
            m = col(4)
            cidx_v[slot, pl.ds(g * L, L)] = (
                (w % 7) * NDAY + ((h % 24) * 60 + m % 60) // 5)

        gather(slot).start()

        @pl.when(k >= 1)
        def _flush_prev():
            gather(slot ^ 1).wait()
            out_copy(k - 1, slot ^ 1).start()

    def pair_step(k2, _):
        chunk_step(2 * k2, 0)
        chunk_step(2 * k2 + 1, 1)
        return ()

    lax.fori_loop(0, NCHUNK // 2, pair_step, (), unroll=False)

    last = NCHUNK - 1
    gather(last & 1).wait()
    out_copy(last, last & 1).start()
    for k in (NCHUNK - 2, NCHUNK - 1):
        out_copy(k, k & 1).wait()


@functools.partial(jax.jit, static_argnames=())
def kernel(TE, day_table, week_table):
    te_flat = TE.reshape(N * 5 // 128, 128).astype(jnp.int32)
    comb = jnp.concatenate(
        [jnp.broadcast_to(week_table[:, None, :], (7, NDAY, TDIM)),
         jnp.broadcast_to(day_table[None, :, :], (7, NDAY, TDIM))],
        axis=-1).reshape(7 * NDAY, 2 * TDIM)
    run = pl.kernel(
        _body,
        out_type=jax.ShapeDtypeStruct((N, 2 * TDIM), jnp.float32),
        mesh=plsc.VectorSubcoreMesh(core_axis_name="c", subcore_axis_name="s"),
        scratch_types=[
            pltpu.VMEM_SHARED((7 * NDAY, 2 * TDIM), jnp.float32),
            pltpu.VMEM((2, TEROWS, 128), jnp.int32),
            pltpu.VMEM((2, CHUNK), jnp.int32),
            pltpu.VMEM((2, CHUNK, 2 * TDIM), jnp.float32),
            pltpu.SemaphoreType.DMA((2,)),
            pltpu.SemaphoreType.DMA((2,)),
            pltpu.SemaphoreType.DMA((2,)),
            pltpu.SemaphoreType.DMA,
        ],
        compiler_params=pltpu.CompilerParams(
            use_tc_tiling_on_sc=False, needs_layout_passes=False),
    )
    out = run(te_flat, comb)
    return out.reshape(B, T, 2 * TDIM)


# speedup vs baseline: 1.0507x; 1.0507x over previous
import functools

import jax
import jax.numpy as jnp
from jax import lax
from jax.experimental import pallas as pl
from jax.experimental.pallas import tpu as pltpu
from jax.experimental.pallas import tpu_sc as plsc

B, T, TDIM = 4096, 200, 64
N = B * T
NC, NS, L = 2, 16, 16
NW = NC * NS
NPW = N // NW
CHUNK = 128
NCHUNK = NPW // CHUNK
GROUPS = CHUNK // L
TEROWS = CHUNK * 5 // 128
NDAY = 288


def _body(te_hbm, comb_hbm, out_hbm,
          comb_t, te_v, cidx_v, rows_v,
          tsem, gsem, osem, lsem):
    sid = lax.axis_index("s")
    wid = sid * NC + lax.axis_index("c")
    lane5 = jax.lax.iota(jnp.int32, L) * 5
    chunk0 = wid * NCHUNK

    @pl.when(sid == 0)
    def _load_table():
        pltpu.async_copy(comb_hbm, comb_t, lsem).wait()

    plsc.subcore_barrier()

    def te_copy(k, slot):
        return pltpu.make_async_copy(
            te_hbm.at[pl.ds((chunk0 + k) * TEROWS, TEROWS)],
            te_v.at[slot], tsem.at[slot])

    def gather(slot):
        return pltpu.make_async_copy(
            comb_t.at[cidx_v.at[slot]], rows_v.at[slot], gsem.at[slot])

    def out_copy(k, slot):
        return pltpu.make_async_copy(
            rows_v.at[slot],
            out_hbm.at[pl.ds((chunk0 + k) * CHUNK, CHUNK)], osem.at[slot])

    te_copy(0, 0).start()

    def chunk_step(k, slot):
        te_copy(k, slot).wait()

        @pl.when(k + 1 < NCHUNK)
        def _prefetch():
            te_copy(k + 1, slot ^ 1).start()

        @pl.when(k >= 2)
        def _drain_out():
            out_copy(k - 2, slot).wait()

        @plsc.parallel_loop(0, GROUPS, unroll=2)
        def group_step(g):
            i5 = lane5 + g * (L * 5)

            def col(off):
                idx = i5 + off
                return plsc.load_gather(te_v.at[slot], [idx >> 7, idx & 127])

            w = col(2)
            h = col(3)
            m = col(4)
            cidx_v[slot, pl.ds(g * L, L)] = (
                (w % 7) * NDAY + ((h % 24) * 60 + m % 60) // 5)

        gather(slot).start()

        @pl.when(k >= 1)
        def _flush_prev():
            gather(slot ^ 1).wait()
            out_copy(k - 1, slot ^ 1).start()

    def pair_step(k2, _):
        chunk_step(2 * k2, 0)
        chunk_step(2 * k2 + 1, 1)
        return ()

    lax.fori_loop(0, NCHUNK // 2, pair_step, (), unroll=False)

    last = NCHUNK - 1
    gather(last & 1).wait()
    out_copy(last, last & 1).start()
    for k in (NCHUNK - 2, NCHUNK - 1):
        out_copy(k, k & 1).wait()


@functools.partial(jax.jit, static_argnames=())
def kernel(TE, day_table, week_table):
    te_flat = TE.reshape(N * 5 // 128, 128).astype(jnp.int32)
    comb = jnp.concatenate(
        [jnp.broadcast_to(week_table[:, None, :], (7, NDAY, TDIM)),
         jnp.broadcast_to(day_table[None, :, :], (7, NDAY, TDIM))],
        axis=-1).reshape(7 * NDAY, 2 * TDIM)
    run = pl.kernel(
        _body,
        out_type=jax.ShapeDtypeStruct((N, 2 * TDIM), jnp.float32),
        mesh=plsc.VectorSubcoreMesh(core_axis_name="c", subcore_axis_name="s"),
        scratch_types=[
            pltpu.VMEM_SHARED((7 * NDAY, 2 * TDIM), jnp.float32),
            pltpu.VMEM((2, TEROWS, 128), jnp.int32),
            pltpu.VMEM((2, CHUNK), jnp.int32),
            pltpu.VMEM((2, CHUNK, 2 * TDIM), jnp.float32),
            pltpu.SemaphoreType.DMA((2,)),
            pltpu.SemaphoreType.DMA((2,)),
            pltpu.SemaphoreType.DMA((2,)),
            pltpu.SemaphoreType.DMA,
        ],
        compiler_params=pltpu.CompilerParams(
            use_tc_tiling_on_sc=False, needs_layout_passes=False),
    )
    out = run(te_flat, comb)
    return out.reshape(B, T, 2 * TDIM)


# speedup vs baseline: 2.8082x; 2.6728x over previous
import functools

import jax
import jax.numpy as jnp
from jax import lax
from jax.experimental import pallas as pl
from jax.experimental.pallas import tpu as pltpu
from jax.experimental.pallas import tpu_sc as plsc

B, T, TDIM = 4096, 200, 64
N = B * T
NC, NS, L = 2, 16, 16
NW = NC * NS
NPW = N // NW
CHUNK = 128
NCHUNK = NPW // CHUNK
GROUPS = CHUNK // L
NDAY = 288


def _body(te_hbm, comb_hbm, out_hbm,
          comb_t, te_v, cidx_v, rows_v,
          tsem, gsem, osem, lsem):
    sid = lax.axis_index("s")
    wid = sid * NC + lax.axis_index("c")
    chunk0 = wid * NCHUNK

    @pl.when(sid == 0)
    def _load_table():
        pltpu.async_copy(comb_hbm, comb_t, lsem).wait()

    plsc.subcore_barrier()

    def te_copy(k, slot):
        return pltpu.make_async_copy(
            te_hbm.at[chunk0 + k], te_v.at[slot], tsem.at[slot])

    def gather(slot):
        return pltpu.make_async_copy(
            comb_t.at[cidx_v.at[slot]], rows_v.at[slot], gsem.at[slot])

    def out_copy(k, slot):
        return pltpu.make_async_copy(
            rows_v.at[slot],
            out_hbm.at[pl.ds((chunk0 + k) * CHUNK, CHUNK)], osem.at[slot])

    te_copy(0, 0).start()

    def chunk_step(k, slot):
        te_copy(k, slot).wait()

        @pl.when(k + 1 < NCHUNK)
        def _prefetch():
            te_copy(k + 1, slot ^ 1).start()

        @pl.when(k >= 2)
        def _drain_out():
            out_copy(k - 2, slot).wait()

        @plsc.parallel_loop(0, GROUPS, unroll=2)
        def group_step(g):
            sl = pl.ds(g * L, L)
            w = te_v[slot, 0, sl]
            h = te_v[slot, 1, sl]
            m = te_v[slot, 2, sl]
            wmod = w - ((w * 18725) >> 17) * 7
            hmod = h - ((h * 21846) >> 19) * 24
            mmod = m - ((m * 17477) >> 20) * 60
            d = hmod * 60 + mmod
            cidx_v[slot, sl] = wmod * NDAY + ((d * 6554) >> 15)

        gather(slot).start()

        @pl.when(k >= 1)
        def _flush_prev():
            gather(slot ^ 1).wait()
            out_copy(k - 1, slot ^ 1).start()

    def pair_step(k2, _):
        chunk_step(2 * k2, 0)
        chunk_step(2 * k2 + 1, 1)
        return ()

    lax.fori_loop(0, NCHUNK // 2, pair_step, (), unroll=False)

    last = NCHUNK - 1
    gather(last & 1).wait()
    out_copy(last, last & 1).start()
    for k in (NCHUNK - 2, NCHUNK - 1):
        out_copy(k, k & 1).wait()


@functools.partial(jax.jit, static_argnames=())
def kernel(TE, day_table, week_table):
    te_cols = jnp.swapaxes(
        TE.reshape(N, 5)[:, 2:5].reshape(N // CHUNK, CHUNK, 3), 1, 2
    ).astype(jnp.int32)
    comb = jnp.concatenate(
        [jnp.broadcast_to(week_table[:, None, :], (7, NDAY, TDIM)),
         jnp.broadcast_to(day_table[None, :, :], (7, NDAY, TDIM))],
        axis=-1).reshape(7 * NDAY, 2 * TDIM)
    run = pl.kernel(
        _body,
        out_type=jax.ShapeDtypeStruct((N, 2 * TDIM), jnp.float32),
        mesh=plsc.VectorSubcoreMesh(core_axis_name="c", subcore_axis_name="s"),
        scratch_types=[
            pltpu.VMEM_SHARED((7 * NDAY, 2 * TDIM), jnp.float32),
            pltpu.VMEM((2, 3, CHUNK), jnp.int32),
            pltpu.VMEM((2, CHUNK), jnp.int32),
            pltpu.VMEM((2, CHUNK, 2 * TDIM), jnp.float32),
            pltpu.SemaphoreType.DMA((2,)),
            pltpu.SemaphoreType.DMA((2,)),
            pltpu.SemaphoreType.DMA((2,)),
            pltpu.SemaphoreType.DMA,
        ],
        compiler_params=pltpu.CompilerParams(
            use_tc_tiling_on_sc=False, needs_layout_passes=False),
    )
    out = run(te_cols, comb)
    return out.reshape(B, T, 2 * TDIM)
